# baseline (speedup 1.0000x reference)
import functools

import jax
import jax.numpy as jnp
from jax import lax
from jax.experimental import pallas as pl
from jax.experimental.pallas import tpu as pltpu
from jax.experimental.pallas import tpu_sc as plsc

N = 10000
E = 320000
D = 128
NC = 2
NS = 16
NW = NC * NS
N_PAD = 10240
RNG = N_PAD // NW
E_PAD = 327680
CHUNK = 2048
NCHUNK = E_PAD // CHUNK
GB = 128
DEG_W = 16


def _agg_body(table_hbm, src_hbm, dst_hbm, out_hbm,
              sbuf, dbuf, gidx, ridx, rows0, rows1, acc, sem0, sem1):
    cid = lax.axis_index("c")
    sid = lax.axis_index("s")
    wid = cid * NS + sid
    base = wid * RNG

    def init_acc(i, carry):
        for j in range(D // 16):
            acc[i, pl.ds(j * 16, 16)] = jnp.zeros((16,), jnp.float32)
        return carry

    lax.fori_loop(0, RNG, init_acc, 0)

    def gather(b, rows, sem):
        idx = plsc.Indices(gidx.at[pl.ds(b * GB, GB)], ignored_value=-1)
        return pltpu.async_copy(table_hbm.at[idx], rows, sem)

    def add_block(b, rows):
        def edge(k, carry2):
            row = ridx[pl.ds(b * GB + k, 16)][0]

            @pl.when(row >= 0)
            def _():
                for j in range(D // 16):
                    v = rows[k, pl.ds(j * 16, 16)]
                    plsc.addupdate(acc.at[row, pl.ds(j * 16, 16)], v)
            return carry2

        lax.fori_loop(0, GB, edge, 0)

    def chunk(c, carry):
        ebase = c * CHUNK
        pltpu.sync_copy(src_hbm.at[pl.ds(ebase, CHUNK)], sbuf)
        pltpu.sync_copy(dst_hbm.at[pl.ds(ebase, CHUNK)], dbuf)
        neg1 = jnp.full((16,), -1, jnp.int32)
        for g in range(CHUNK // 16):
            dvec = dbuf[pl.ds(g * 16, 16)]
            svec = sbuf[pl.ds(g * 16, 16)]
            dl = dvec - base
            m = (dvec >= base) & (dl < RNG)
            gidx[pl.ds(g * 16, 16)] = jnp.where(m, svec, neg1)
            ridx[pl.ds(g * 16, 16)] = jnp.where(m, dl, neg1)
        bufs = (rows0, rows1)
        sems = (sem0, sem1)
        nblk = CHUNK // GB
        gather(0, rows0, sem0)
        for b in range(nblk):
            if b + 1 < nblk:
                gather(b + 1, bufs[(b + 1) % 2], sems[(b + 1) % 2])
            idx = plsc.Indices(gidx.at[pl.ds(b * GB, GB)], ignored_value=-1)
            pltpu.make_async_copy(table_hbm.at[idx], bufs[b % 2],
                                  sems[b % 2]).wait()
            add_block(b, bufs[b % 2])
        return carry

    lax.fori_loop(0, NCHUNK, chunk, 0)
    pltpu.sync_copy(
        acc, out_hbm.at[pl.ds(pl.multiple_of(base, 8), RNG)])


@functools.cache
def _sc_kernels():
    mesh = plsc.VectorSubcoreMesh(
        core_axis_name="c", subcore_axis_name="s",
        num_cores=NC, num_subcores=NS)
    return pl.kernel(
        _agg_body,
        out_type=jax.ShapeDtypeStruct((N_PAD, D), jnp.float32),
        mesh=mesh,
        scratch_types=[
            pltpu.VMEM((CHUNK,), jnp.int32),
            pltpu.VMEM((CHUNK,), jnp.int32),
            pltpu.VMEM((CHUNK,), jnp.int32),
            pltpu.VMEM((CHUNK + 16,), jnp.int32),
            pltpu.VMEM((GB, D), jnp.float32),
            pltpu.VMEM((GB, D), jnp.float32),
            pltpu.VMEM((RNG, D), jnp.float32),
            pltpu.SemaphoreType.DMA,
            pltpu.SemaphoreType.DMA,
        ],
    )


BR = 256
GR = N_PAD // BR


def _dinv(deg_ref):
    return lax.rsqrt(deg_ref[:, 0:1] + 1.0)


def _rowmask(i):
    return i * BR + lax.broadcasted_iota(jnp.int32, (BR, 1), 0) < N


def _a_body(x_ref, w_ref, deg_ref, o_ref):
    i = pl.program_id(0)
    h = jnp.dot(x_ref[...], w_ref[...], preferred_element_type=jnp.float32)
    o_ref[...] = jnp.where(_rowmask(i), h * _dinv(deg_ref), 0.0)


def _b1_body(agg_ref, hp_ref, deg_ref, b1_ref, o_ref):
    a = agg_ref[...] + hp_ref[...]
    o_ref[...] = jnp.maximum(a * _dinv(deg_ref) + b1_ref[...], 0.0)


def _b2_body(agg_ref, hp_ref, deg_ref, b2_ref, o_ref):
    a = agg_ref[...] + hp_ref[...]
    o_ref[...] = a * _dinv(deg_ref) + b2_ref[...]


_spec_rows = pl.BlockSpec((BR, D), lambda i: (i, 0))
_spec_w = pl.BlockSpec((D, D), lambda i: (0, 0))
_spec_b = pl.BlockSpec((1, D), lambda i: (0, 0))


def kernel(x, edge_index, W1, b1, W2, b2):
    src = edge_index[0].astype(jnp.int32)
    dst = edge_index[1].astype(jnp.int32)
    pad = jnp.full((E_PAD - E,), N, jnp.int32)
    src_p = jnp.concatenate([src, pad])
    dst_p = jnp.concatenate([dst, pad])
    xp = jnp.concatenate([x, jnp.zeros((N_PAD - N, D), jnp.float32)])

    agg_kernel = _sc_kernels()

    ones_tab = jnp.concatenate([jnp.ones((N, D), jnp.float32),
                                jnp.zeros((N_PAD - N, D), jnp.float32)])
    deg = agg_kernel(ones_tab, src_p, dst_p)

    hp1 = pl.pallas_call(
        _a_body,
        grid=(GR,),
        in_specs=[_spec_rows, _spec_w, _spec_rows],
        out_specs=_spec_rows,
        out_shape=jax.ShapeDtypeStruct((N_PAD, D), jnp.float32),
    )(xp, W1, deg)

    agg1 = agg_kernel(hp1, src_p, dst_p)

    h = pl.pallas_call(
        _b1_body,
        grid=(GR,),
        in_specs=[_spec_rows, _spec_rows, _spec_rows, _spec_b],
        out_specs=_spec_rows,
        out_shape=jax.ShapeDtypeStruct((N_PAD, D), jnp.float32),
    )(agg1, hp1, deg, b1.reshape(1, D))

    hp2 = pl.pallas_call(
        _a_body,
        grid=(GR,),
        in_specs=[_spec_rows, _spec_w, _spec_rows],
        out_specs=_spec_rows,
        out_shape=jax.ShapeDtypeStruct((N_PAD, D), jnp.float32),
    )(h, W2, deg)

    agg2 = agg_kernel(hp2, src_p, dst_p)

    out = pl.pallas_call(
        _b2_body,
        grid=(GR,),
        in_specs=[_spec_rows, _spec_rows, _spec_rows, _spec_b],
        out_specs=_spec_rows,
        out_shape=jax.ShapeDtypeStruct((N, D), jnp.float32),
    )(agg2, hp2, deg, b2.reshape(1, D))
    return out


# speedup vs baseline: 2.7189x; 2.7189x over previous
import functools

import jax
import jax.numpy as jnp
from jax import lax
from jax.experimental import pallas as pl
from jax.experimental.pallas import tpu as pltpu
from jax.experimental.pallas import tpu_sc as plsc

N = 10000
E = 320000
D = 128
NC = 2
NS = 16
NW = NC * NS
N_PAD = 10240
RNG = N_PAD // NW
E_PAD = 327680
CHUNK = 2048
NCHUNK = E_PAD // CHUNK
GB = 128
DEG_W = 16


def _agg_body(table_hbm, src_hbm, dst_hbm, out_hbm,
              sbuf, dbuf, gidx, ridx, rows0, rows1, acc, sem0, sem1):
    cid = lax.axis_index("c")
    sid = lax.axis_index("s")
    wid = cid * NS + sid
    base = wid * RNG

    def init_acc(i, carry):
        for j in range(D // 16):
            acc[i, pl.ds(j * 16, 16)] = jnp.zeros((16,), jnp.float32)
        return carry

    lax.fori_loop(0, RNG, init_acc, 0)

    def gather(b, rows, sem):
        idx = plsc.Indices(gidx.at[pl.ds(b * GB, GB)], ignored_value=-1)
        return pltpu.async_copy(table_hbm.at[idx], rows, sem)

    def add_block(b, rows):
        def group(g, carry2):
            rvec = ridx[pl.ds(b * GB + g * 16, 16)]
            rws = [rvec[k] for k in range(16)]
            anym = rws[0]
            for k in range(1, 16):
                anym = jnp.maximum(anym, rws[k])

            @pl.when(anym >= 0)
            def _():
                for k in range(16):
                    row = rws[k]

                    @pl.when(row >= 0)
                    def _():
                        for j in range(D // 16):
                            v = rows[g * 16 + k, pl.ds(j * 16, 16)]
                            plsc.addupdate(
                                acc.at[row, pl.ds(j * 16, 16)], v)
            return carry2

        lax.fori_loop(0, GB // 16, group, 0)

    def chunk(c, carry):
        ebase = c * CHUNK
        pltpu.sync_copy(src_hbm.at[pl.ds(ebase, CHUNK)], sbuf)
        pltpu.sync_copy(dst_hbm.at[pl.ds(ebase, CHUNK)], dbuf)
        neg1 = jnp.full((16,), -1, jnp.int32)
        for g in range(CHUNK // 16):
            dvec = dbuf[pl.ds(g * 16, 16)]
            svec = sbuf[pl.ds(g * 16, 16)]
            dl = dvec - base
            m = (dvec >= base) & (dl < RNG)
            gidx[pl.ds(g * 16, 16)] = jnp.where(m, svec, neg1)
            ridx[pl.ds(g * 16, 16)] = jnp.where(m, dl, neg1)
        bufs = (rows0, rows1)
        sems = (sem0, sem1)
        nblk = CHUNK // GB
        gather(0, rows0, sem0)
        for b in range(nblk):
            if b + 1 < nblk:
                gather(b + 1, bufs[(b + 1) % 2], sems[(b + 1) % 2])
            idx = plsc.Indices(gidx.at[pl.ds(b * GB, GB)], ignored_value=-1)
            pltpu.make_async_copy(table_hbm.at[idx], bufs[b % 2],
                                  sems[b % 2]).wait()
            add_block(b, bufs[b % 2])
        return carry

    lax.fori_loop(0, NCHUNK, chunk, 0)
    pltpu.sync_copy(
        acc, out_hbm.at[pl.ds(pl.multiple_of(base, 8), RNG)])


@functools.cache
def _sc_kernels():
    mesh = plsc.VectorSubcoreMesh(
        core_axis_name="c", subcore_axis_name="s",
        num_cores=NC, num_subcores=NS)
    return pl.kernel(
        _agg_body,
        out_type=jax.ShapeDtypeStruct((N_PAD, D), jnp.float32),
        mesh=mesh,
        scratch_types=[
            pltpu.VMEM((CHUNK,), jnp.int32),
            pltpu.VMEM((CHUNK,), jnp.int32),
            pltpu.VMEM((CHUNK,), jnp.int32),
            pltpu.VMEM((CHUNK + 16,), jnp.int32),
            pltpu.VMEM((GB, D), jnp.float32),
            pltpu.VMEM((GB, D), jnp.float32),
            pltpu.VMEM((RNG, D), jnp.float32),
            pltpu.SemaphoreType.DMA,
            pltpu.SemaphoreType.DMA,
        ],
    )


BR = 256
GR = N_PAD // BR


def _dinv(deg_ref):
    return lax.rsqrt(deg_ref[:, 0:1] + 1.0)


def _rowmask(i):
    return i * BR + lax.broadcasted_iota(jnp.int32, (BR, 1), 0) < N


def _a_body(x_ref, w_ref, deg_ref, o_ref):
    i = pl.program_id(0)
    h = jnp.dot(x_ref[...], w_ref[...], preferred_element_type=jnp.float32)
    o_ref[...] = jnp.where(_rowmask(i), h * _dinv(deg_ref), 0.0)


def _b1_body(agg_ref, hp_ref, deg_ref, b1_ref, o_ref):
    a = agg_ref[...] + hp_ref[...]
    o_ref[...] = jnp.maximum(a * _dinv(deg_ref) + b1_ref[...], 0.0)


def _b2_body(agg_ref, hp_ref, deg_ref, b2_ref, o_ref):
    a = agg_ref[...] + hp_ref[...]
    o_ref[...] = a * _dinv(deg_ref) + b2_ref[...]


_spec_rows = pl.BlockSpec((BR, D), lambda i: (i, 0))
_spec_w = pl.BlockSpec((D, D), lambda i: (0, 0))
_spec_b = pl.BlockSpec((1, D), lambda i: (0, 0))


def kernel(x, edge_index, W1, b1, W2, b2):
    src = edge_index[0].astype(jnp.int32)
    dst = edge_index[1].astype(jnp.int32)
    pad = jnp.full((E_PAD - E,), N, jnp.int32)
    src_p = jnp.concatenate([src, pad])
    dst_p = jnp.concatenate([dst, pad])
    xp = jnp.concatenate([x, jnp.zeros((N_PAD - N, D), jnp.float32)])

    agg_kernel = _sc_kernels()

    ones_tab = jnp.concatenate([jnp.ones((N, D), jnp.float32),
                                jnp.zeros((N_PAD - N, D), jnp.float32)])
    deg = agg_kernel(ones_tab, src_p, dst_p)

    hp1 = pl.pallas_call(
        _a_body,
        grid=(GR,),
        in_specs=[_spec_rows, _spec_w, _spec_rows],
        out_specs=_spec_rows,
        out_shape=jax.ShapeDtypeStruct((N_PAD, D), jnp.float32),
    )(xp, W1, deg)

    agg1 = agg_kernel(hp1, src_p, dst_p)

    h = pl.pallas_call(
        _b1_body,
        grid=(GR,),
        in_specs=[_spec_rows, _spec_rows, _spec_rows, _spec_b],
        out_specs=_spec_rows,
        out_shape=jax.ShapeDtypeStruct((N_PAD, D), jnp.float32),
    )(agg1, hp1, deg, b1.reshape(1, D))

    hp2 = pl.pallas_call(
        _a_body,
        grid=(GR,),
        in_specs=[_spec_rows, _spec_w, _spec_rows],
        out_specs=_spec_rows,
        out_shape=jax.ShapeDtypeStruct((N_PAD, D), jnp.float32),
    )(h, W2, deg)

    agg2 = agg_kernel(hp2, src_p, dst_p)

    out = pl.pallas_call(
        _b2_body,
        grid=(GR,),
        in_specs=[_spec_rows, _spec_rows, _spec_rows, _spec_b],
        out_specs=_spec_rows,
        out_shape=jax.ShapeDtypeStruct((N, D), jnp.float32),
    )(agg2, hp2, deg, b2.reshape(1, D))
    return out
